# Initial kernel scaffold; baseline (speedup 1.0000x reference)
import jax
import jax.numpy as jnp
from jax.experimental import pallas as pl


def kernel(regressions, anchors_concat, annotations, class_id):
    raise NotImplementedError("write your pallas kernel here")


# speedup vs baseline: 3.2626x; 3.2626x over previous
import functools

import numpy as np
import jax
import jax.numpy as jnp
from jax import lax
from jax.experimental import pallas as pl
from jax.experimental.pallas import tpu as pltpu
from jax.experimental.pallas import tpu_sc as plsc

_LEVEL_SIZES = (32768, 16384, 8192, 4096, 2048, 1024)
_TOTAL = sum(_LEVEL_SIZES)
_NUM_GT = 32
_B = 8
_NC = 2
_NS = 16
_W = _NC * _NS
_CHUNK = _TOTAL // _W
_NV = _CHUNK // 16
_LANES = 16
_RCH = _CHUNK * 2


def _aux_arrays():
    rate = 22050.0 / 256.0
    sizes = [x * rate for x in
             [2.23147392, 2.62519274, 3.74199546, 5.78800454, 8.02371882, np.inf]]
    lower = np.zeros(_TOTAL, np.float32)
    upper = np.zeros(_TOTAL, np.float32)
    inv = np.zeros(_TOTAL, np.float32)
    s = 0
    for i, n in enumerate(_LEVEL_SIZES):
        lower[s:s + n] = np.float32(sizes[i - 1] if i > 0 else 0.0)
        upper[s:s + n] = np.float32(sizes[i])
        inv[s:s + n] = np.float32(1.0 / (2 ** i))
        s += n
    return jnp.asarray(lower), jnp.asarray(upper), jnp.asarray(inv)


def _body(anch_hbm, low_hbm, up_hbm, inv_hbm, reg0_hbm, reg1_hbm, s_hbm,
          e_hbm, c_hbm, cls_hbm, out_hbm,
          anch_v, low_v, up_v, inv_v, reg0_v, reg1_v, s_v, e_v, c_v, cls_v,
          key_v, acc_v):
    wid = lax.axis_index("s") * _NC + lax.axis_index("c")
    a0 = wid * _CHUNK
    pltpu.sync_copy(anch_hbm.at[pl.ds(a0, _CHUNK)], anch_v)
    pltpu.sync_copy(low_hbm.at[pl.ds(a0, _CHUNK)], low_v)
    pltpu.sync_copy(up_hbm.at[pl.ds(a0, _CHUNK)], up_v)
    pltpu.sync_copy(inv_hbm.at[pl.ds(a0, _CHUNK)], inv_v)
    pltpu.sync_copy(s_hbm, s_v)
    pltpu.sync_copy(e_hbm, e_v)
    pltpu.sync_copy(c_hbm, c_v)
    pltpu.sync_copy(cls_hbm, cls_v)
    for b in range(_B):
        pltpu.sync_copy(reg0_hbm.at[pl.ds(b * _TOTAL + a0, _CHUNK)],
                        reg0_v.at[pl.ds(b * _CHUNK, _CHUNK)])
        pltpu.sync_copy(reg1_hbm.at[pl.ds(b * _TOTAL + a0, _CHUNK)],
                        reg1_v.at[pl.ds(b * _CHUNK, _CHUNK)])

    inf = jnp.float32(np.inf)
    clsv = cls_v[...]
    for b in range(_B):
        for h in range(_NUM_GT // _LANES):
            sl = pl.ds(b * _NUM_GT + h * _LANES, _LANES)
            key_v[sl] = jnp.where(c_v[sl] == clsv, e_v[sl] - s_v[sl], inf)

    zero = jnp.zeros((_LANES,), jnp.float32)
    for b in range(_B):
        nh = _NUM_GT // _LANES
        sh = [s_v[pl.ds(b * _NUM_GT + h * _LANES, _LANES)] for h in range(nh)]
        eh = [e_v[pl.ds(b * _NUM_GT + h * _LANES, _LANES)] for h in range(nh)]
        kh = [key_v[pl.ds(b * _NUM_GT + h * _LANES, _LANES)] for h in range(nh)]
        svals = [sh[g // _LANES][g % _LANES] for g in range(_NUM_GT)]
        evals = [eh[g // _LANES][g % _LANES] for g in range(_NUM_GT)]
        kvals = [kh[g // _LANES][g % _LANES] for g in range(_NUM_GT)]

        def vbody(v, carry, svals=svals, evals=evals, kvals=kvals, b=b):
            acc, cnt = carry
            base = v * _LANES
            sl = pl.ds(base, _LANES)
            a = anch_v[sl]
            lo = low_v[sl]
            up = up_v[sl]
            iv = inv_v[sl]
            bk = jnp.full((_LANES,), inf, jnp.float32)
            infv = bk
            tl = zero
            tr = zero
            for g in range(_NUM_GT):
                l = a - svals[g]
                r = evals[g] - a
                mn = jnp.minimum(l, r)
                mx = jnp.maximum(l, r)
                m1 = jnp.minimum(mn, mx - lo)
                v1 = jnp.where(m1 >= 0.0, kvals[g], inf)
                cand = jnp.where(mx < up, v1, inf)
                better = cand < bk
                bk = jnp.minimum(bk, cand)
                tl = jnp.where(better, l, tl)
                tr = jnp.where(better, r, tr)
            pos = bk < infv
            rsl = pl.ds(b * _CHUNK + base, _LANES)
            r0 = reg0_v[rsl]
            r1 = reg1_v[rsl]
            d = jnp.abs(tl * iv - r0) + jnp.abs(tr * iv - r1)
            acc = acc + jnp.where(pos, d, 0.0)
            cnt = cnt + jnp.where(pos, 1.0, 0.0)
            return acc, cnt

        acc, cnt = lax.fori_loop(0, _NV, vbody, (zero, zero))
        acc_v[pl.ds(b * 2 * _LANES, _LANES)] = acc
        acc_v[pl.ds(b * 2 * _LANES + _LANES, _LANES)] = cnt

    pltpu.sync_copy(acc_v, out_hbm.at[pl.ds(wid * _B * 2 * _LANES,
                                            _B * 2 * _LANES)])


@functools.cache
def _launcher():
    mesh = plsc.VectorSubcoreMesh(core_axis_name="c", subcore_axis_name="s")
    return pl.kernel(
        _body,
        mesh=mesh,
        out_type=jax.ShapeDtypeStruct((_W * _B * 2 * _LANES,), jnp.float32),
        scratch_types=[
            pltpu.VMEM((_CHUNK,), jnp.float32),
            pltpu.VMEM((_CHUNK,), jnp.float32),
            pltpu.VMEM((_CHUNK,), jnp.float32),
            pltpu.VMEM((_CHUNK,), jnp.float32),
            pltpu.VMEM((_B * _CHUNK,), jnp.float32),
            pltpu.VMEM((_B * _CHUNK,), jnp.float32),
            pltpu.VMEM((_B * _NUM_GT,), jnp.float32),
            pltpu.VMEM((_B * _NUM_GT,), jnp.float32),
            pltpu.VMEM((_B * _NUM_GT,), jnp.float32),
            pltpu.VMEM((_LANES,), jnp.float32),
            pltpu.VMEM((_B * _NUM_GT,), jnp.float32),
            pltpu.VMEM((_B * 2 * _LANES,), jnp.float32),
        ],
    )


def kernel(regressions, anchors_concat, annotations, class_id):
    low, up, inv = _aux_arrays()
    reg0 = regressions[:, :, 0].reshape(-1)
    reg1 = regressions[:, :, 1].reshape(-1)
    s_a = annotations[:, :, 0].reshape(-1)
    e_a = annotations[:, :, 1].reshape(-1)
    c_a = annotations[:, :, 2].reshape(-1)
    cls16 = jnp.full((_LANES,), jnp.asarray(class_id, jnp.float32))
    parts = _launcher()(anchors_concat, low, up, inv, reg0, reg1, s_a, e_a,
                        c_a, cls16)
    parts = parts.reshape(_W, _B, 2, _LANES)
    sums = jnp.sum(parts[:, :, 0, :], axis=(0, 2))
    cnts = jnp.sum(parts[:, :, 1, :], axis=(0, 2))
    loss = jnp.where(cnts > 0.0, sums / (jnp.maximum(cnts, 1.0) * 2.0), 0.0)
    return jnp.mean(loss, keepdims=True)


# speedup vs baseline: 10.5294x; 3.2273x over previous
import functools

import numpy as np
import jax
import jax.numpy as jnp
from jax import lax
from jax.experimental import pallas as pl
from jax.experimental.pallas import tpu as pltpu
from jax.experimental.pallas import tpu_sc as plsc

_LEVEL_SIZES = (32768, 16384, 8192, 4096, 2048, 1024)
_TOTAL = sum(_LEVEL_SIZES)
_NUM_GT = 32
_B = 8
_NC = 2
_NS = 16
_W = _NC * _NS
_CHUNK = _TOTAL // _W
_LANES = 16

_SEG_N = tuple(n // _W for n in _LEVEL_SIZES)
_SEG_BASE = tuple(int(x) for x in np.cumsum((0,) + _SEG_N[:-1]))
_LVL_OFF = tuple(int(x) for x in np.cumsum((0,) + _LEVEL_SIZES[:-1]))

_RATE = 22050.0 / 256.0
_SIZES = tuple(x * _RATE for x in
               (2.23147392, 2.62519274, 3.74199546, 5.78800454, 8.02371882,
                np.inf))
_LO = tuple((_SIZES[i - 1] if i > 0 else 0.0) for i in range(6))
_UP = _SIZES
_INV = tuple(1.0 / (2 ** i) for i in range(6))


def _body(anch_hbm, reg0_hbm, reg1_hbm, s_hbm, e_hbm, c_hbm, cls_hbm, out_hbm,
          anch_v, reg0_v, reg1_v, srep_v, erep_v, crep_v, cls_v, key_v,
          bk_v, tl_v, tr_v, acc_v, semA, semR):
    wid = lax.axis_index("s") * _NC + lax.axis_index("c")
    handles_a = []
    for i in range(6):
        src = anch_hbm.at[pl.ds(_LVL_OFF[i] + wid * _SEG_N[i], _SEG_N[i])]
        dst = anch_v.at[pl.ds(_SEG_BASE[i], _SEG_N[i])]
        handles_a.append(pltpu.async_copy(src, dst, semA))
    handles_a.append(pltpu.async_copy(s_hbm, srep_v, semA))
    handles_a.append(pltpu.async_copy(e_hbm, erep_v, semA))
    handles_a.append(pltpu.async_copy(c_hbm, crep_v, semA))
    handles_a.append(pltpu.async_copy(cls_hbm, cls_v, semA))
    handles_r = []
    for i in range(6):
        for b in range(_B):
            off = b * _TOTAL + _LVL_OFF[i] + wid * _SEG_N[i]
            dst = pl.ds(b * _CHUNK + _SEG_BASE[i], _SEG_N[i])
            handles_r.append(pltpu.async_copy(
                reg0_hbm.at[pl.ds(off, _SEG_N[i])], reg0_v.at[dst], semR))
            handles_r.append(pltpu.async_copy(
                reg1_hbm.at[pl.ds(off, _SEG_N[i])], reg1_v.at[dst], semR))
    for h in handles_a:
        h.wait()

    inf = jnp.float32(np.inf)
    infv = jnp.full((_LANES,), inf, jnp.float32)
    zero = jnp.zeros((_LANES,), jnp.float32)

    def initb(i, _):
        bk_v[pl.ds(i * _LANES, _LANES)] = infv
        return 0
    lax.fori_loop(0, _B * _CHUNK // _LANES, initb, 0)

    clsv = cls_v[...]

    def keyb(i, _):
        sl = pl.ds(i * _LANES, _LANES)
        key_v[sl] = jnp.where(crep_v[sl] == clsv, erep_v[sl] - srep_v[sl], inf)
        return 0
    lax.fori_loop(0, _B * _NUM_GT, keyb, 0)

    for seg in range(6):
        m = _SEG_N[seg]
        sb = _SEG_BASE[seg]
        nv = m // _LANES
        lo = jnp.float32(_LO[seg])
        up = jnp.float32(_UP[seg])
        up2 = jnp.float32(2.0 * _UP[seg] if np.isfinite(_UP[seg]) else np.inf)
        amin = anch_v[pl.ds(sb, _LANES)][0]
        amax = anch_v[pl.ds(sb + m - _LANES, _LANES)][_LANES - 1]

        def gbody(gb, _, sb=sb, nv=nv, lo=lo, up=up, up2=up2,
                  amin=amin, amax=amax):
            g = gb // _B
            b = gb % _B
            sl = pl.ds((b * _NUM_GT + g) * _LANES, _LANES)
            s16 = srep_v[sl]
            e16 = erep_v[sl]
            k16 = key_v[sl]
            s_s = s16[0]
            e_s = e16[0]
            k_s = k16[0]
            t = jnp.minimum(jnp.minimum(e_s - amin, amax - s_s),
                            jnp.minimum(k_s - lo, up2 - k_s))

            @pl.when(t >= 0.0)
            def _():
                def vb(v, _):
                    asl = pl.ds(sb + v * _LANES, _LANES)
                    ssl = pl.ds(b * _CHUNK + sb + v * _LANES, _LANES)
                    a = anch_v[asl]
                    bk = bk_v[ssl]
                    tl = tl_v[ssl]
                    tr = tr_v[ssl]
                    l = a - s16
                    r = e16 - a
                    mn = jnp.minimum(l, r)
                    mx = jnp.maximum(l, r)
                    m1 = jnp.minimum(mn, mx - lo)
                    v1 = jnp.where(m1 >= 0.0, k16, inf)
                    cand = jnp.where(mx < up, v1, inf)
                    better = cand < bk
                    bk_v[ssl] = jnp.minimum(bk, cand)
                    tl_v[ssl] = jnp.where(better, l, tl)
                    tr_v[ssl] = jnp.where(better, r, tr)
                    return 0
                lax.fori_loop(0, nv, vb, 0)
            return 0
        lax.fori_loop(0, _NUM_GT * _B, gbody, 0)

    for h in handles_r:
        h.wait()

    def fb(b, _):
        acc = zero
        cnt = zero
        for seg in range(6):
            nv = _SEG_N[seg] // _LANES
            sb = _SEG_BASE[seg]
            iv = jnp.float32(_INV[seg])

            def vb(v, carry, sb=sb, iv=iv, b=b):
                acc, cnt = carry
                ssl = pl.ds(b * _CHUNK + sb + v * _LANES, _LANES)
                bk = bk_v[ssl]
                tl = tl_v[ssl]
                tr = tr_v[ssl]
                r0 = reg0_v[ssl]
                r1 = reg1_v[ssl]
                pos = bk < inf
                d = jnp.abs(tl * iv - r0) + jnp.abs(tr * iv - r1)
                acc = acc + jnp.where(pos, d, 0.0)
                cnt = cnt + jnp.where(pos, 1.0, 0.0)
                return acc, cnt
            acc, cnt = lax.fori_loop(0, nv, vb, (acc, cnt))
        acc_v[pl.ds(b * 2 * _LANES, _LANES)] = acc
        acc_v[pl.ds(b * 2 * _LANES + _LANES, _LANES)] = cnt
        return 0
    lax.fori_loop(0, _B, fb, 0)

    pltpu.sync_copy(acc_v, out_hbm.at[pl.ds(wid * _B * 2 * _LANES,
                                            _B * 2 * _LANES)])


@functools.cache
def _launcher():
    mesh = plsc.VectorSubcoreMesh(core_axis_name="c", subcore_axis_name="s")
    nrep = _B * _NUM_GT * _LANES
    return pl.kernel(
        _body,
        mesh=mesh,
        out_type=jax.ShapeDtypeStruct((_W * _B * 2 * _LANES,), jnp.float32),
        scratch_types=[
            pltpu.VMEM((_CHUNK,), jnp.float32),
            pltpu.VMEM((_B * _CHUNK,), jnp.float32),
            pltpu.VMEM((_B * _CHUNK,), jnp.float32),
            pltpu.VMEM((nrep,), jnp.float32),
            pltpu.VMEM((nrep,), jnp.float32),
            pltpu.VMEM((nrep,), jnp.float32),
            pltpu.VMEM((_LANES,), jnp.float32),
            pltpu.VMEM((nrep,), jnp.float32),
            pltpu.VMEM((_B * _CHUNK,), jnp.float32),
            pltpu.VMEM((_B * _CHUNK,), jnp.float32),
            pltpu.VMEM((_B * _CHUNK,), jnp.float32),
            pltpu.VMEM((_B * 2 * _LANES,), jnp.float32),
            pltpu.SemaphoreType.DMA,
            pltpu.SemaphoreType.DMA,
        ],
    )


def kernel(regressions, anchors_concat, annotations, class_id):
    reg0 = regressions[:, :, 0].reshape(-1)
    reg1 = regressions[:, :, 1].reshape(-1)
    rep = (_B, _NUM_GT, _LANES)
    s_a = jnp.broadcast_to(annotations[:, :, 0:1], rep).reshape(-1)
    e_a = jnp.broadcast_to(annotations[:, :, 1:2], rep).reshape(-1)
    c_a = jnp.broadcast_to(annotations[:, :, 2:3], rep).reshape(-1)
    cls16 = jnp.full((_LANES,), jnp.asarray(class_id, jnp.float32))
    parts = _launcher()(anchors_concat, reg0, reg1, s_a, e_a, c_a, cls16)
    parts = parts.reshape(_W, _B, 2, _LANES)
    sums = jnp.sum(parts[:, :, 0, :], axis=(0, 2))
    cnts = jnp.sum(parts[:, :, 1, :], axis=(0, 2))
    loss = jnp.where(cnts > 0.0, sums / (jnp.maximum(cnts, 1.0) * 2.0), 0.0)
    return jnp.mean(loss, keepdims=True)


# speedup vs baseline: 16.3406x; 1.5519x over previous
import functools

import numpy as np
import jax
import jax.numpy as jnp
from jax import lax
from jax.experimental import pallas as pl
from jax.experimental.pallas import tpu as pltpu
from jax.experimental.pallas import tpu_sc as plsc

_LEVEL_SIZES = (32768, 16384, 8192, 4096, 2048, 1024)
_TOTAL = sum(_LEVEL_SIZES)
_NUM_GT = 32
_B = 8
_NC = 2
_NS = 16
_W = _NC * _NS
_CHUNK = _TOTAL // _W
_LANES = 16

_SEG_N = tuple(n // _W for n in _LEVEL_SIZES)
_SEG_BASE = tuple(int(x) for x in np.cumsum((0,) + _SEG_N[:-1]))
_LVL_OFF = tuple(int(x) for x in np.cumsum((0,) + _LEVEL_SIZES[:-1]))

_RATE = 22050.0 / 256.0
_SIZES = tuple(x * _RATE for x in
               (2.23147392, 2.62519274, 3.74199546, 5.78800454, 8.02371882,
                np.inf))
_LO = tuple((_SIZES[i - 1] if i > 0 else 0.0) for i in range(6))
_UP = _SIZES
_INV = tuple(1.0 / (2 ** i) for i in range(6))


def _body(anch_hbm, reg0_hbm, reg1_hbm, s_hbm, e_hbm, c_hbm, su_hbm, eu_hbm,
          cu_hbm, cls_hbm, out_hbm,
          anch_v, reg0_v, reg1_v, srep_v, erep_v, crep_v, su_v, eu_v, cu_v,
          cls_v, key_v, keyu_v, bk_v, tl_v, tr_v, acc_v, semA, semR):
    wid = lax.axis_index("s") * _NC + lax.axis_index("c")
    handles_a = []
    for i in range(6):
        src = anch_hbm.at[pl.ds(_LVL_OFF[i] + wid * _SEG_N[i], _SEG_N[i])]
        dst = anch_v.at[pl.ds(_SEG_BASE[i], _SEG_N[i])]
        handles_a.append(pltpu.async_copy(src, dst, semA))
    handles_a.append(pltpu.async_copy(s_hbm, srep_v, semA))
    handles_a.append(pltpu.async_copy(e_hbm, erep_v, semA))
    handles_a.append(pltpu.async_copy(c_hbm, crep_v, semA))
    handles_a.append(pltpu.async_copy(su_hbm, su_v, semA))
    handles_a.append(pltpu.async_copy(eu_hbm, eu_v, semA))
    handles_a.append(pltpu.async_copy(cu_hbm, cu_v, semA))
    handles_a.append(pltpu.async_copy(cls_hbm, cls_v, semA))
    handles_r = []
    for i in range(6):
        for b in range(_B):
            off = b * _TOTAL + _LVL_OFF[i] + wid * _SEG_N[i]
            dst = pl.ds(b * _CHUNK + _SEG_BASE[i], _SEG_N[i])
            handles_r.append(pltpu.async_copy(
                reg0_hbm.at[pl.ds(off, _SEG_N[i])], reg0_v.at[dst], semR))
            handles_r.append(pltpu.async_copy(
                reg1_hbm.at[pl.ds(off, _SEG_N[i])], reg1_v.at[dst], semR))
    for h in handles_a:
        h.wait()

    inf = jnp.float32(np.inf)
    infv = jnp.full((_LANES,), inf, jnp.float32)
    zero = jnp.zeros((_LANES,), jnp.float32)

    def initb(i, _):
        bk_v[pl.ds(i * _LANES, _LANES)] = infv
        return 0
    lax.fori_loop(0, _B * _CHUNK // _LANES, initb, 0)

    clsv = cls_v[...]

    def keyb(i, _):
        sl = pl.ds(i * _LANES, _LANES)
        key_v[sl] = jnp.where(crep_v[sl] == clsv, erep_v[sl] - srep_v[sl], inf)
        return 0
    lax.fori_loop(0, _B * _NUM_GT, keyb, 0)

    def keyu(i, _):
        sl = pl.ds(i * _LANES, _LANES)
        keyu_v[sl] = jnp.where(cu_v[sl] == clsv, eu_v[sl] - su_v[sl], inf)
        return 0
    lax.fori_loop(0, _B * _NUM_GT // _LANES, keyu, 0)

    iot = lax.broadcasted_iota(jnp.int32, (_LANES,), 0)
    perms = [jnp.bitwise_xor(iot, jnp.int32(sh)) for sh in (8, 4, 2, 1)]

    def _lane_min(x):
        for p in perms:
            x = jnp.minimum(x, x.at[p].get(mode="promise_in_bounds"))
        return x[0]

    def _lane_sum(x):
        for p in perms:
            x = x + x.at[p].get(mode="promise_in_bounds")
        return x[0]

    for seg in range(6):
        m = _SEG_N[seg]
        sb = _SEG_BASE[seg]
        nv = m // _LANES
        lo = jnp.float32(_LO[seg])
        up = jnp.float32(_UP[seg])
        up2 = jnp.float32(2.0 * _UP[seg] if np.isfinite(_UP[seg]) else np.inf)
        amin = anch_v[pl.ds(sb, _LANES)][0]
        amax = anch_v[pl.ds(sb + m - _LANES, _LANES)][_LANES - 1]

        def bbody(b, _, sb=sb, nv=nv, lo=lo, up=up, up2=up2,
                  amin=amin, amax=amax):
            for h in range(_NUM_GT // _LANES):
                usl = pl.ds(b * _NUM_GT + h * _LANES, _LANES)
                sv = su_v[usl]
                ev = eu_v[usl]
                kv = keyu_v[usl]
                t = jnp.minimum(jnp.minimum(ev - amin, amax - sv),
                                jnp.minimum(kv - lo, up2 - kv))
                fidx = jnp.where(t >= 0.0, iot, jnp.int32(_LANES))
                cnt = _lane_sum(jnp.where(t >= 0.0, jnp.int32(1),
                                          jnp.int32(0)))

                def wbody(_, gprev, fidx=fidx, b=b, h=h, sb=sb, nv=nv,
                          lo=lo, up=up):
                    g = _lane_min(jnp.where(iot > gprev, fidx,
                                            jnp.int32(_LANES)))
                    sl = pl.ds((b * _NUM_GT + h * _LANES + g) * _LANES,
                               _LANES)
                    s16 = srep_v[sl]
                    e16 = erep_v[sl]
                    k16 = key_v[sl]

                    def vb(v, _):
                        asl = pl.ds(sb + v * _LANES, _LANES)
                        ssl = pl.ds(b * _CHUNK + sb + v * _LANES, _LANES)
                        a = anch_v[asl]
                        bk = bk_v[ssl]
                        tl = tl_v[ssl]
                        tr = tr_v[ssl]
                        l = a - s16
                        r = e16 - a
                        mn = jnp.minimum(l, r)
                        mx = jnp.maximum(l, r)
                        m1 = jnp.minimum(mn, mx - lo)
                        v1 = jnp.where(m1 >= 0.0, k16, inf)
                        cand = jnp.where(mx < up, v1, inf)
                        better = cand < bk
                        bk_v[ssl] = jnp.minimum(bk, cand)
                        tl_v[ssl] = jnp.where(better, l, tl)
                        tr_v[ssl] = jnp.where(better, r, tr)
                        return 0
                    lax.fori_loop(0, nv, vb, 0)
                    return g
                lax.fori_loop(0, cnt, wbody, jnp.int32(-1))
            return 0
        lax.fori_loop(0, _B, bbody, 0)

    for h in handles_r:
        h.wait()

    def fb(b, _):
        acc = zero
        cnt = zero
        for seg in range(6):
            nv = _SEG_N[seg] // _LANES
            sb = _SEG_BASE[seg]
            iv = jnp.float32(_INV[seg])

            def vb(v, carry, sb=sb, iv=iv, b=b):
                acc, cnt = carry
                ssl = pl.ds(b * _CHUNK + sb + v * _LANES, _LANES)
                bk = bk_v[ssl]
                tl = tl_v[ssl]
                tr = tr_v[ssl]
                r0 = reg0_v[ssl]
                r1 = reg1_v[ssl]
                pos = bk < inf
                d = jnp.abs(tl * iv - r0) + jnp.abs(tr * iv - r1)
                acc = acc + jnp.where(pos, d, 0.0)
                cnt = cnt + jnp.where(pos, 1.0, 0.0)
                return acc, cnt
            acc, cnt = lax.fori_loop(0, nv, vb, (acc, cnt))
        acc_v[pl.ds(b * 2 * _LANES, _LANES)] = acc
        acc_v[pl.ds(b * 2 * _LANES + _LANES, _LANES)] = cnt
        return 0
    lax.fori_loop(0, _B, fb, 0)

    pltpu.sync_copy(acc_v, out_hbm.at[pl.ds(wid * _B * 2 * _LANES,
                                            _B * 2 * _LANES)])


@functools.cache
def _launcher():
    mesh = plsc.VectorSubcoreMesh(core_axis_name="c", subcore_axis_name="s")
    nrep = _B * _NUM_GT * _LANES
    return pl.kernel(
        _body,
        mesh=mesh,
        out_type=jax.ShapeDtypeStruct((_W * _B * 2 * _LANES,), jnp.float32),
        scratch_types=[
            pltpu.VMEM((_CHUNK,), jnp.float32),
            pltpu.VMEM((_B * _CHUNK,), jnp.float32),
            pltpu.VMEM((_B * _CHUNK,), jnp.float32),
            pltpu.VMEM((nrep,), jnp.float32),
            pltpu.VMEM((nrep,), jnp.float32),
            pltpu.VMEM((nrep,), jnp.float32),
            pltpu.VMEM((_B * _NUM_GT,), jnp.float32),
            pltpu.VMEM((_B * _NUM_GT,), jnp.float32),
            pltpu.VMEM((_B * _NUM_GT,), jnp.float32),
            pltpu.VMEM((_LANES,), jnp.float32),
            pltpu.VMEM((nrep,), jnp.float32),
            pltpu.VMEM((_B * _NUM_GT,), jnp.float32),
            pltpu.VMEM((_B * _CHUNK,), jnp.float32),
            pltpu.VMEM((_B * _CHUNK,), jnp.float32),
            pltpu.VMEM((_B * _CHUNK,), jnp.float32),
            pltpu.VMEM((_B * 2 * _LANES,), jnp.float32),
            pltpu.SemaphoreType.DMA,
            pltpu.SemaphoreType.DMA,
        ],
    )


def kernel(regressions, anchors_concat, annotations, class_id):
    reg0 = regressions[:, :, 0].reshape(-1)
    reg1 = regressions[:, :, 1].reshape(-1)
    rep = (_B, _NUM_GT, _LANES)
    s_a = jnp.broadcast_to(annotations[:, :, 0:1], rep).reshape(-1)
    e_a = jnp.broadcast_to(annotations[:, :, 1:2], rep).reshape(-1)
    c_a = jnp.broadcast_to(annotations[:, :, 2:3], rep).reshape(-1)
    s_u = annotations[:, :, 0].reshape(-1)
    e_u = annotations[:, :, 1].reshape(-1)
    c_u = annotations[:, :, 2].reshape(-1)
    cls16 = jnp.full((_LANES,), jnp.asarray(class_id, jnp.float32))
    parts = _launcher()(anchors_concat, reg0, reg1, s_a, e_a, c_a, s_u, e_u,
                        c_u, cls16)
    parts = parts.reshape(_W, _B, 2, _LANES)
    sums = jnp.sum(parts[:, :, 0, :], axis=(0, 2))
    cnts = jnp.sum(parts[:, :, 1, :], axis=(0, 2))
    loss = jnp.where(cnts > 0.0, sums / (jnp.maximum(cnts, 1.0) * 2.0), 0.0)
    return jnp.mean(loss, keepdims=True)
